# Initial kernel scaffold; baseline (speedup 1.0000x reference)
import jax
import jax.numpy as jnp
from jax.experimental import pallas as pl


def kernel(x, edge_index, batch, W1, b1, W2, b2, W3, b3, W4, b4, W5, b5, W6, b6, W7, b7, Wo, bo):
    raise NotImplementedError("write your pallas kernel here")


# speedup vs baseline: 13.6608x; 13.6608x over previous
import functools

import jax
import jax.numpy as jnp
from jax import lax
from jax.experimental import pallas as pl
from jax.experimental.pallas import tpu as pltpu
from jax.experimental.pallas import tpu_sc as plsc

N = 10000
NP = 10240
G = 16
E = 320000
EP = 327680
EB = 128
NW = 32
RPW = EP // EB // NW
RPT = NP // 16


def _make_agg(w):
    mesh = plsc.VectorSubcoreMesh(core_axis_name="c", subcore_axis_name="s")

    @functools.partial(
        pl.kernel,
        mesh=mesh,
        out_type=jax.ShapeDtypeStruct((2 * NP, w), jnp.float32),
        compiler_params=pltpu.CompilerParams(use_tc_tiling_on_sc=False),
        scratch_types=[
            pltpu.VMEM((RPW, EB), jnp.int32),
            pltpu.VMEM((RPW, EB), jnp.int32),
            pltpu.VMEM((EB, w), jnp.float32),
            pltpu.VMEM_SHARED((NP, w), jnp.float32),
            pltpu.SemaphoreType.DMA,
        ],
    )
    def agg(g_hbm, src_hbm, dst_hbm, out_hbm, src_v, dst_v, rows_v, acc, sem):
        cid = lax.axis_index("c")
        sid = lax.axis_index("s")
        wid = cid * 16 + sid
        r0 = sid * RPT
        pltpu.sync_copy(g_hbm.at[pl.ds(r0, RPT)], acc.at[pl.ds(r0, RPT)])
        base = wid * RPW
        pltpu.sync_copy(src_hbm.at[pl.ds(base, RPW)], src_v)
        pltpu.sync_copy(dst_hbm.at[pl.ds(base, RPW)], dst_v)
        plsc.subcore_barrier()

        def body(j, carry):
            pltpu.async_copy(g_hbm.at[src_v.at[j]], rows_v, sem).wait()
            pltpu.sync_copy(rows_v, acc.at[dst_v.at[j]], add=True)
            return carry

        lax.fori_loop(0, RPW, body, 0)
        plsc.subcore_barrier()
        pltpu.sync_copy(acc.at[pl.ds(r0, RPT)],
                        out_hbm.at[pl.ds(cid * NP + r0, RPT)])

    return agg


def _tc(body, out_shapes, *args):
    return pl.pallas_call(body, out_shape=out_shapes)(*args)


def _first_body(degp_ref, x_ref, w1_ref, g1_ref, dinv_ref):
    deg = degp_ref[0:NP, 0:1] + degp_ref[NP:2 * NP, 0:1] - 1.0
    dinv = lax.rsqrt(jnp.maximum(deg, 1.0))
    dinv_ref[...] = dinv
    g1_ref[...] = jnp.dot(x_ref[...], w1_ref[...],
                          preferred_element_type=jnp.float32) * dinv


def _make_mid(prev_pre, next_pre):
    def body(*refs):
        it = iter(refs)
        p_ref = next(it)
        g_ref = next(it)
        dinv_ref = next(it)
        wp_ref = next(it) if prev_pre else None
        bp_ref = next(it)
        wn_ref = None if next_pre else next(it)
        out_ref = next(it)
        s = p_ref[0:NP, :] + p_ref[NP:2 * NP, :] - g_ref[...]
        dinv = dinv_ref[...]
        t = dinv * s
        if prev_pre:
            t = jnp.dot(t, wp_ref[...], preferred_element_type=jnp.float32)
        h = jnp.maximum(t + bp_ref[...], 0.0)
        if next_pre:
            out_ref[...] = dinv * h
        else:
            out_ref[...] = dinv * jnp.dot(
                h, wn_ref[...], preferred_element_type=jnp.float32)
    return body


def _final_body(p_ref, g_ref, dinv_ref, b7_ref, batch_ref, wo_ref, bo_ref,
                out_ref):
    s = p_ref[0:NP, :] + p_ref[NP:2 * NP, :] - g_ref[...]
    h = jnp.maximum(dinv_ref[...] * s + b7_ref[...], 0.0)
    onehot = (batch_ref[...] ==
              lax.broadcasted_iota(jnp.int32, (1, G), 1)).astype(jnp.float32)
    dnums = (((0,), (0,)), ((), ()))
    counts = lax.dot_general(onehot, jnp.ones((NP, 1), jnp.float32), dnums,
                             preferred_element_type=jnp.float32)
    pooled = lax.dot_general(onehot, h, dnums,
                             preferred_element_type=jnp.float32)
    pooled = pooled / jnp.maximum(counts, 1.0)
    out_ref[...] = jnp.dot(pooled, wo_ref[...],
                           preferred_element_type=jnp.float32) + bo_ref[...]


def kernel(x, edge_index, batch, W1, b1, W2, b2, W3, b3, W4, b4, W5, b5,
           W6, b6, W7, b7, Wo, bo):
    f32 = jnp.float32
    xp = jnp.pad(x, ((0, NP - N), (0, 0)))
    src = jnp.concatenate(
        [edge_index[0], jnp.zeros((EP - E,), jnp.int32)]).reshape(NW * RPW, EB)
    dst = jnp.concatenate(
        [edge_index[1], jnp.full((EP - E,), N, jnp.int32)]).reshape(NW * RPW, EB)
    batch_p = jnp.pad(batch, (0, NP - N), constant_values=G).reshape(NP, 1)
    ones16 = jnp.ones((NP, 16), f32)

    agg16 = _make_agg(16)
    agg32 = _make_agg(32)
    agg64 = _make_agg(64)

    degp = agg16(ones16, src, dst)

    g1, dinv = _tc(
        _first_body,
        (jax.ShapeDtypeStruct((NP, 16), f32),
         jax.ShapeDtypeStruct((NP, 1), f32)),
        degp, xp, W1)

    p = agg16(g1, src, dst)
    g2 = _tc(_make_mid(False, True), jax.ShapeDtypeStruct((NP, 16), f32),
             p, g1, dinv, b1.reshape(1, -1))
    p = agg16(g2, src, dst)
    g3 = _tc(_make_mid(True, True), jax.ShapeDtypeStruct((NP, 32), f32),
             p, g2, dinv, W2, b2.reshape(1, -1))
    p = agg32(g3, src, dst)
    g4 = _tc(_make_mid(True, True), jax.ShapeDtypeStruct((NP, 64), f32),
             p, g3, dinv, W3, b3.reshape(1, -1))
    p = agg64(g4, src, dst)
    g5 = _tc(_make_mid(True, False), jax.ShapeDtypeStruct((NP, 64), f32),
             p, g4, dinv, W4, b4.reshape(1, -1), W5)
    p = agg64(g5, src, dst)
    g6 = _tc(_make_mid(False, False), jax.ShapeDtypeStruct((NP, 32), f32),
             p, g5, dinv, b5.reshape(1, -1), W6)
    p = agg32(g6, src, dst)
    g7 = _tc(_make_mid(False, False), jax.ShapeDtypeStruct((NP, 16), f32),
             p, g6, dinv, b6.reshape(1, -1), W7)
    p = agg16(g7, src, dst)
    out = _tc(_final_body, jax.ShapeDtypeStruct((G, 1), f32),
              p, g7, dinv, b7.reshape(1, -1), batch_p, Wo,
              bo.reshape(1, 1))
    return jnp.squeeze(out)
